# baseline (speedup 1.0000x reference)
import jax
import jax.numpy as jnp
from jax import lax
from jax.experimental import pallas as pl
from jax.experimental.pallas import tpu as pltpu
from jax.experimental.pallas import tpu_sc as plsc

VOCAB = 100000
ROWS = 4096
CHUNK = 16
NCH = 4
BATCH = 16384

NC = 2
NS = 16
NW = NC * NS
B_PER_W = BATCH // NW
E_PER_W = B_PER_W * NCH
GCH = 128
NB = B_PER_W // GCH
R_PER_T = ROWS // NS
HB = 64
NSLAB = 3


def _body(x_hbm, h0_hbm, h1_hbm, t0_hbm, t1_hbm, out_hbm,
          xv, e0, h0v, h1v, slab0, slab1, slab2, tbufA, tbufB,
          ts0, ts1, g0, g1, sem,
          semS0, semS1, semS2, semP0, semP1):
    sid = lax.axis_index("s")
    wid = sid * NC + lax.axis_index("c")
    base_b = wid * B_PER_W

    with jax.named_scope("p_setup"):
        pltpu.sync_copy(x_hbm.at[pl.ds(base_b, B_PER_W)], xv)

        @plsc.parallel_loop(0, NCH * (B_PER_W // 16), unroll=4)
        def _(j):
            c = lax.shift_right_logical(j, 5)
            i = lax.bitwise_and(j, 31)
            e0[c, pl.ds(i * 16, 16)] = xv[pl.ds(i * 16, 16)] + c * VOCAB

        h_copies = []
        for c in range(NCH):
            for b in range(NB):
                sl = pl.ds(b * GCH, GCH)
                h_copies.append(pltpu.async_copy(
                    h0_hbm.at[e0.at[c, sl]], h0v.at[c, sl], sem))
                h_copies.append(pltpu.async_copy(
                    h1_hbm.at[e0.at[c, sl]], h1v.at[c, sl], sem))

    iota = lax.iota(jnp.int32, 16)
    nhalf = R_PER_T // HB
    chunks = [(t, h) for t in range(2) for h in range(nhalf)]
    slabs = (slab0, slab1, slab2)
    semS = (semS0, semS1, semS2)
    tbufs = (tbufA, tbufB)
    semP = (semP0, semP1)

    def start_stage(i):
        tbl, half = chunks[i]
        col0 = sid * R_PER_T + half * HB
        return pltpu.async_copy(
            (t0_hbm, t1_hbm)[tbl].at[:, pl.ds(col0, HB)],
            slabs[i % NSLAB], semS[i % NSLAB])

    with jax.named_scope("p_stage"):
        stage_h = [start_stage(i) for i in range(NSLAB - 1)]
        push_h = []
        for i, (tbl, half) in enumerate(chunks):
            col0 = sid * R_PER_T + half * HB
            if i + NSLAB - 1 < len(chunks):
                stage_h.append(start_stage(i + NSLAB - 1))
            stage_h[i].wait()
            if i >= 2:
                for ph in push_h[(i - 2) * NCH:(i - 1) * NCH]:
                    ph.wait()
            slab, tbuf = slabs[i % NSLAB], tbufs[i % 2]
            for c in range(NCH):
                @plsc.parallel_loop(0, CHUNK, unroll=4)
                def _(s):
                    fs = jnp.full((16,), s, jnp.int32)
                    for q in range(HB // 16):
                        v = slab[c * CHUNK + s, pl.ds(q * 16, 16)]
                        plsc.store_scatter(
                            tbuf, [c * HB + q * 16 + iota, fs], v)

            ts = (ts0, ts1)[tbl]
            for c in range(NCH):
                push_h.append(pltpu.async_copy(
                    tbuf.at[pl.ds(c * HB, HB), pl.ds(0, CHUNK)],
                    ts.at[pl.ds(c * ROWS + col0, HB)], semP[i % 2]))
        for ph in push_h[(len(chunks) - 2) * NCH:]:
            ph.wait()

    with jax.named_scope("p_hwait"):
        for cp in h_copies:
            cp.wait()

        @plsc.parallel_loop(0, NCH * (B_PER_W // 16), unroll=4)
        def _(j):
            c = lax.shift_right_logical(j, 5)
            i = lax.bitwise_and(j, 31)
            sl = pl.ds(i * 16, 16)
            h0v[c, sl] = h0v[c, sl] + c * ROWS
            h1v[c, sl] = h1v[c, sl] + c * ROWS

    with jax.named_scope("p_bar"):
        plsc.subcore_barrier()

    with jax.named_scope("p_tgather"):
        t_copies = []
        for c in range(NCH):
            for b in range(NB):
                sl = pl.ds(b * GCH, GCH)
                row0 = (c * NB + b) * GCH
                t_copies.append(pltpu.async_copy(
                    ts0.at[h0v.at[c, sl]], g0.at[pl.ds(row0, GCH)], sem))
                t_copies.append(pltpu.async_copy(
                    ts1.at[h1v.at[c, sl]], g1.at[pl.ds(row0, GCH)], sem))
        for cp in t_copies:
            cp.wait()

    with jax.named_scope("p_accum"):
        @plsc.parallel_loop(0, E_PER_W, unroll=8)
        def _(i):
            g0[i, :] = g0[i, :] + g1[i, :]

    with jax.named_scope("p_out"):
        out_h = [pltpu.async_copy(
            g0.at[pl.ds(c * B_PER_W, B_PER_W)],
            out_hbm.at[pl.ds(base_b, B_PER_W), pl.ds(c * CHUNK, CHUNK)],
            sem) for c in range(NCH)]
        for oh in out_h:
            oh.wait()


@jax.jit
def _cc_embed(x, h0t, h1t, t0, t1):
    mesh = plsc.VectorSubcoreMesh(core_axis_name="c", subcore_axis_name="s")
    kfn = pl.kernel(
        _body,
        out_type=jax.ShapeDtypeStruct((BATCH, NCH * CHUNK), jnp.float32),
        mesh=mesh,
        compiler_params=pltpu.CompilerParams(
            needs_layout_passes=False, use_tc_tiling_on_sc=False),
        scratch_types=[
            pltpu.VMEM((B_PER_W,), jnp.int32),
            pltpu.VMEM((NCH, B_PER_W), jnp.int32),
            pltpu.VMEM((NCH, B_PER_W), jnp.int32),
            pltpu.VMEM((NCH, B_PER_W), jnp.int32),
            pltpu.VMEM((NCH * CHUNK, HB), jnp.float32),
            pltpu.VMEM((NCH * CHUNK, HB), jnp.float32),
            pltpu.VMEM((NCH * CHUNK, HB), jnp.float32),
            pltpu.VMEM((NCH * HB, CHUNK + 1), jnp.float32),
            pltpu.VMEM((NCH * HB, CHUNK + 1), jnp.float32),
            pltpu.VMEM_SHARED((NCH * ROWS, CHUNK), jnp.float32),
            pltpu.VMEM_SHARED((NCH * ROWS, CHUNK), jnp.float32),
            pltpu.VMEM((E_PER_W, CHUNK), jnp.float32),
            pltpu.VMEM((E_PER_W, CHUNK), jnp.float32),
            pltpu.SemaphoreType.DMA,
            pltpu.SemaphoreType.DMA,
            pltpu.SemaphoreType.DMA,
            pltpu.SemaphoreType.DMA,
            pltpu.SemaphoreType.DMA,
            pltpu.SemaphoreType.DMA,
        ],
    )
    return kfn(x, h0t, h1t, t0, t1)


def kernel(x, table0, table1, h0, h1):
    h0t = h0.T.reshape(VOCAB * NCH)
    h1t = h1.T.reshape(VOCAB * NCH)
    t0 = table0.transpose(1, 2, 0).reshape(NCH * CHUNK, ROWS)
    t1 = table1.transpose(1, 2, 0).reshape(NCH * CHUNK, ROWS)
    return _cc_embed(x.astype(jnp.int32), h0t, h1t, t0, t1)


# speedup vs baseline: 1.0330x; 1.0330x over previous
import jax
import jax.numpy as jnp
from jax import lax
from jax.experimental import pallas as pl
from jax.experimental.pallas import tpu as pltpu
from jax.experimental.pallas import tpu_sc as plsc

VOCAB = 100000
ROWS = 4096
CHUNK = 16
NCH = 4
BATCH = 16384

NC = 2
NS = 16
NW = NC * NS
B_PER_W = BATCH // NW
GCH = 128
NB = B_PER_W // GCH
R_PER_T = ROWS // NS
HB = 128
NSLAB = 3


def _body(x_hbm, h0_hbm, h1_hbm, t0_hbm, t1_hbm, out_hbm,
          xv, e0, h0v, h1v, slab0, slab1, slab2, tbufA, tbufB,
          ts0, ts1, gA0, gA1, gB0, gB1, sem,
          semS0, semS1, semS2, semP0, semP1, semT0, semT1, semO0, semO1):
    sid = lax.axis_index("s")
    wid = sid * NC + lax.axis_index("c")
    base_b = wid * B_PER_W

    with jax.named_scope("p_setup"):
        pltpu.sync_copy(x_hbm.at[pl.ds(base_b, B_PER_W)], xv)

        @plsc.parallel_loop(0, NCH * (B_PER_W // 16), unroll=4)
        def _(j):
            c = lax.shift_right_logical(j, 5)
            i = lax.bitwise_and(j, 31)
            e0[c, pl.ds(i * 16, 16)] = xv[pl.ds(i * 16, 16)] + c * VOCAB

        h_copies = []
        for c in range(NCH):
            for b in range(NB):
                sl = pl.ds(b * GCH, GCH)
                h_copies.append(pltpu.async_copy(
                    h0_hbm.at[e0.at[c, sl]], h0v.at[c, sl], sem))
                h_copies.append(pltpu.async_copy(
                    h1_hbm.at[e0.at[c, sl]], h1v.at[c, sl], sem))

    iota = lax.iota(jnp.int32, 16)
    nhalf = R_PER_T // HB
    chunks = [(t, h) for t in range(2) for h in range(nhalf)]
    slabs = (slab0, slab1, slab2)
    semS = (semS0, semS1, semS2)
    tbufs = (tbufA, tbufB)
    semP = (semP0, semP1)

    def start_stage(i):
        tbl, half = chunks[i]
        col0 = sid * R_PER_T + half * HB
        return pltpu.async_copy(
            (t0_hbm, t1_hbm)[tbl].at[:, pl.ds(col0, HB)],
            slabs[i % NSLAB], semS[i % NSLAB])

    with jax.named_scope("p_stage"):
        stage_h = [start_stage(i) for i in range(NSLAB - 1)]
        push_h = []
        for i, (tbl, half) in enumerate(chunks):
            col0 = sid * R_PER_T + half * HB
            if i + NSLAB - 1 < len(chunks):
                stage_h.append(start_stage(i + NSLAB - 1))
            stage_h[i].wait()
            if i >= 2:
                for ph in push_h[(i - 2) * NCH:(i - 1) * NCH]:
                    ph.wait()
            slab, tbuf = slabs[i % NSLAB], tbufs[i % 2]
            for c in range(NCH):
                @plsc.parallel_loop(0, CHUNK, unroll=2)
                def _(s):
                    fs = jnp.full((16,), s, jnp.int32)
                    for q in range(HB // 16):
                        v = slab[c * CHUNK + s, pl.ds(q * 16, 16)]
                        plsc.store_scatter(
                            tbuf, [c * HB + q * 16 + iota, fs], v)

            ts = (ts0, ts1)[tbl]
            for c in range(NCH):
                push_h.append(pltpu.async_copy(
                    tbuf.at[pl.ds(c * HB, HB), pl.ds(0, CHUNK)],
                    ts.at[pl.ds(c * ROWS + col0, HB)], semP[i % 2]))
        for ph in push_h[(len(chunks) - 2) * NCH:]:
            ph.wait()

    with jax.named_scope("p_hwait"):
        for cp in h_copies:
            cp.wait()

        @plsc.parallel_loop(0, NCH * (B_PER_W // 16), unroll=4)
        def _(j):
            c = lax.shift_right_logical(j, 5)
            i = lax.bitwise_and(j, 31)
            sl = pl.ds(i * 16, 16)
            h0v[c, sl] = h0v[c, sl] + c * ROWS
            h1v[c, sl] = h1v[c, sl] + c * ROWS

    with jax.named_scope("p_bar"):
        plsc.subcore_barrier()

    g0b = (gA0, gB0)
    g1b = (gA1, gB1)
    semT = (semT0, semT1)
    semO = (semO0, semO1)

    def fire_gathers(c, p):
        hs = []
        for b in range(NB):
            sl = pl.ds(b * GCH, GCH)
            dst = pl.ds(b * GCH, GCH)
            hs.append(pltpu.async_copy(
                ts0.at[h0v.at[c, sl]], g0b[p].at[dst], semT[p]))
            hs.append(pltpu.async_copy(
                ts1.at[h1v.at[c, sl]], g1b[p].at[dst], semT[p]))
        return hs

    with jax.named_scope("p_tail"):
        out_h = [None, None]
        gh = fire_gathers(0, 0)
        for c in range(NCH):
            p = c % 2
            if c + 1 < NCH:
                if out_h[1 - p] is not None:
                    out_h[1 - p].wait()
                gh_next = fire_gathers(c + 1, 1 - p)
            for h in gh:
                h.wait()

            @plsc.parallel_loop(0, B_PER_W, unroll=8)
            def _(i):
                g0b[p][i, :] = g0b[p][i, :] + g1b[p][i, :]

            out_h[p] = pltpu.async_copy(
                g0b[p],
                out_hbm.at[pl.ds(base_b, B_PER_W), pl.ds(c * CHUNK, CHUNK)],
                semO[p])
            if c + 1 < NCH:
                gh = gh_next
        for oh in out_h:
            if oh is not None:
                oh.wait()


@jax.jit
def _cc_embed(x, h0t, h1t, t0, t1):
    mesh = plsc.VectorSubcoreMesh(core_axis_name="c", subcore_axis_name="s")
    kfn = pl.kernel(
        _body,
        out_type=jax.ShapeDtypeStruct((BATCH, NCH * CHUNK), jnp.float32),
        mesh=mesh,
        compiler_params=pltpu.CompilerParams(
            needs_layout_passes=False, use_tc_tiling_on_sc=False),
        scratch_types=[
            pltpu.VMEM((B_PER_W,), jnp.int32),
            pltpu.VMEM((NCH, B_PER_W), jnp.int32),
            pltpu.VMEM((NCH, B_PER_W), jnp.int32),
            pltpu.VMEM((NCH, B_PER_W), jnp.int32),
            pltpu.VMEM((NCH * CHUNK, HB), jnp.float32),
            pltpu.VMEM((NCH * CHUNK, HB), jnp.float32),
            pltpu.VMEM((NCH * CHUNK, HB), jnp.float32),
            pltpu.VMEM((NCH * HB, CHUNK + 1), jnp.float32),
            pltpu.VMEM((NCH * HB, CHUNK + 1), jnp.float32),
            pltpu.VMEM_SHARED((NCH * ROWS, CHUNK), jnp.float32),
            pltpu.VMEM_SHARED((NCH * ROWS, CHUNK), jnp.float32),
            pltpu.VMEM((B_PER_W, CHUNK), jnp.float32),
            pltpu.VMEM((B_PER_W, CHUNK), jnp.float32),
            pltpu.VMEM((B_PER_W, CHUNK), jnp.float32),
            pltpu.VMEM((B_PER_W, CHUNK), jnp.float32),
            pltpu.SemaphoreType.DMA,
            pltpu.SemaphoreType.DMA,
            pltpu.SemaphoreType.DMA,
            pltpu.SemaphoreType.DMA,
            pltpu.SemaphoreType.DMA,
            pltpu.SemaphoreType.DMA,
            pltpu.SemaphoreType.DMA,
            pltpu.SemaphoreType.DMA,
            pltpu.SemaphoreType.DMA,
            pltpu.SemaphoreType.DMA,
        ],
    )
    return kfn(x, h0t, h1t, t0, t1)


def kernel(x, table0, table1, h0, h1):
    h0t = h0.T.reshape(VOCAB * NCH)
    h1t = h1.T.reshape(VOCAB * NCH)
    t0 = table0.transpose(1, 2, 0).reshape(NCH * CHUNK, ROWS)
    t1 = table1.transpose(1, 2, 0).reshape(NCH * CHUNK, ROWS)
    return _cc_embed(x.astype(jnp.int32), h0t, h1t, t0, t1)
